# baseline (speedup 1.0000x reference)
import functools

import jax
import jax.numpy as jnp
from jax import lax
from jax.experimental import pallas as pl
from jax.experimental.pallas import tpu as pltpu
from jax.experimental.pallas import tpu_sc as plsc

_NC = 2
_NS = 16
_NW = _NC * _NS
_NBUF = 6
_SLACK = 1


def _build(batch, seq, hidden, chunk):
    n_per_w = batch * seq // _NW
    n_ch = n_per_w // chunk
    w_per_b = _NW // batch
    mesh = plsc.VectorSubcoreMesh(core_axis_name="c", subcore_axis_name="s")

    @functools.partial(
        pl.kernel,
        mesh=mesh,
        out_type=jax.ShapeDtypeStruct((batch, seq, hidden), jnp.float32),
        scratch_types=(
            [pltpu.VMEM((n_per_w,), jnp.int32)]
            + [pltpu.VMEM((chunk, hidden), jnp.float32) for _ in range(_NBUF)]
            + [pltpu.SemaphoreType.DMA for _ in range(2 * _NBUF)]
        ),
    )
    def emb(idx_hbm, table_hbm, out_hbm, idx_v, *rest):
        bufs = rest[:_NBUF]
        gsems = rest[_NBUF:2 * _NBUF]
        wsems = rest[2 * _NBUF:]
        wid = lax.axis_index("s") * _NC + lax.axis_index("c")
        bb = wid // w_per_b
        col = (wid % w_per_b) * n_per_w
        pltpu.sync_copy(idx_hbm.at[bb, pl.ds(col, n_per_w)], idx_v)

        def start_gather(i):
            return pltpu.async_copy(table_hbm.at[idx_v.at[pl.ds(i * chunk, chunk)]],
                                    bufs[i % _NBUF], gsems[i % _NBUF])

        lookahead = _NBUF - 1 - _SLACK
        gathers = {j: start_gather(j) for j in range(min(lookahead, n_ch))}
        writebacks = {}
        for i in range(n_ch):
            b = i % _NBUF
            gathers.pop(i).wait()
            writebacks[i] = pltpu.async_copy(
                bufs[b], out_hbm.at[bb, pl.ds(col + i * chunk, chunk)], wsems[b])
            j = i + lookahead
            if j < n_ch:
                if i - 1 - _SLACK in writebacks:
                    writebacks.pop(i - 1 - _SLACK).wait()
                gathers[j] = start_gather(j)
        for i in sorted(writebacks):
            writebacks[i].wait()

    return emb


def kernel(input, word_embeddings):
    b, s = input.shape
    v, d = word_embeddings.shape
    idx = input.astype(jnp.int32)
    return _build(b, s, d, 16)(idx, word_embeddings)


# speedup vs baseline: 1.0028x; 1.0028x over previous
import functools

import jax
import jax.numpy as jnp
from jax import lax
from jax.experimental import pallas as pl
from jax.experimental.pallas import tpu as pltpu
from jax.experimental.pallas import tpu_sc as plsc

_NC = 2
_NS = 16
_NW = _NC * _NS
_NBUF = 6
_SLACK = 0


def _build(batch, seq, hidden, chunk):
    n_per_w = batch * seq // _NW
    n_ch = n_per_w // chunk
    w_per_b = _NW // batch
    mesh = plsc.VectorSubcoreMesh(core_axis_name="c", subcore_axis_name="s")

    @functools.partial(
        pl.kernel,
        mesh=mesh,
        out_type=jax.ShapeDtypeStruct((batch, seq, hidden), jnp.float32),
        scratch_types=(
            [pltpu.VMEM((n_per_w,), jnp.int32),
             pltpu.VMEM((_NBUF * chunk, hidden), jnp.float32)]
            + [pltpu.SemaphoreType.DMA for _ in range(_NBUF + 3)]
        ),
    )
    def emb(idx_hbm, table_hbm, out_hbm, idx_v, buf, *sems):
        gsems = sems[:_NBUF]
        wsems = sems[_NBUF:]
        wid = lax.axis_index("s") * _NC + lax.axis_index("c")
        bb = wid // w_per_b
        col = (wid % w_per_b) * n_per_w
        pltpu.sync_copy(idx_hbm.at[bb, pl.ds(col, n_per_w)], idx_v)

        def start_gather(i):
            return pltpu.async_copy(
                table_hbm.at[idx_v.at[pl.ds(i * chunk, chunk)]],
                buf.at[pl.ds((i % _NBUF) * chunk, chunk)], gsems[i % _NBUF])

        lookahead = _NBUF - 2
        gathers = {j: start_gather(j) for j in range(min(lookahead, n_ch))}
        writebacks = {}
        for i in range(n_ch):
            gathers.pop(i).wait()
            if i % 2 == 1:
                k = i // 2
                writebacks[k] = pltpu.async_copy(
                    buf.at[pl.ds(((i - 1) % _NBUF) * chunk, 2 * chunk)],
                    out_hbm.at[bb, pl.ds(col + (i - 1) * chunk, 2 * chunk)],
                    wsems[k % 3])
            j = i + lookahead
            if j < n_ch:
                if j >= _NBUF:
                    k_need = (j - _NBUF) // 2
                    if k_need in writebacks:
                        writebacks.pop(k_need).wait()
                gathers[j] = start_gather(j)
        for k in sorted(writebacks):
            writebacks[k].wait()

    return emb


def kernel(input, word_embeddings):
    b, s = input.shape
    v, d = word_embeddings.shape
    idx = input.astype(jnp.int32)
    return _build(b, s, d, 16)(idx, word_embeddings)


# speedup vs baseline: 1.0068x; 1.0040x over previous
import functools

import jax
import jax.numpy as jnp
from jax import lax
from jax.experimental import pallas as pl
from jax.experimental.pallas import tpu as pltpu
from jax.experimental.pallas import tpu_sc as plsc

_NC = 2
_NS = 16
_NW = _NC * _NS
_NBUF = 6
_SLACK = 0


def _build(batch, seq, hidden, chunk):
    n_per_w = batch * seq // _NW
    n_ch = n_per_w // chunk
    w_per_b = _NW // batch
    mesh = plsc.VectorSubcoreMesh(core_axis_name="c", subcore_axis_name="s")

    @functools.partial(
        pl.kernel,
        mesh=mesh,
        out_type=jax.ShapeDtypeStruct((batch, seq, hidden), jnp.float32),
        scratch_types=(
            [pltpu.VMEM((n_per_w,), jnp.int32)]
            + [pltpu.VMEM((chunk, hidden), jnp.float32) for _ in range(_NBUF)]
            + [pltpu.SemaphoreType.DMA for _ in range(2 * _NBUF)]
        ),
    )
    def emb(idx_hbm, table_hbm, out_hbm, idx_v, *rest):
        bufs = rest[:_NBUF]
        gsems = rest[_NBUF:2 * _NBUF]
        wsems = rest[2 * _NBUF:]
        wid = lax.axis_index("s") * _NC + lax.axis_index("c")
        bb = wid // w_per_b
        col = (wid % w_per_b) * n_per_w
        pltpu.sync_copy(idx_hbm.at[bb, pl.ds(col, n_per_w)], idx_v)

        def start_gather(i):
            return pltpu.async_copy(table_hbm.at[idx_v.at[pl.ds(i * chunk, chunk)]],
                                    bufs[i % _NBUF], gsems[i % _NBUF])

        lookahead = _NBUF - 1 - _SLACK
        gathers = {j: start_gather(j) for j in range(min(lookahead, n_ch))}
        writebacks = {}
        for i in range(n_ch):
            b = i % _NBUF
            gathers.pop(i).wait()
            writebacks[i] = pltpu.async_copy(
                bufs[b], out_hbm.at[bb, pl.ds(col + i * chunk, chunk)], wsems[b])
            j = i + lookahead
            if j < n_ch:
                if i - 1 - _SLACK in writebacks:
                    writebacks.pop(i - 1 - _SLACK).wait()
                gathers[j] = start_gather(j)
        for i in sorted(writebacks):
            writebacks[i].wait()

    return emb


def kernel(input, word_embeddings):
    b, s = input.shape
    v, d = word_embeddings.shape
    idx = input.astype(jnp.int32)
    return _build(b, s, d, 16)(idx, word_embeddings)


# speedup vs baseline: 1.0087x; 1.0019x over previous
import functools

import jax
import jax.numpy as jnp
from jax import lax
from jax.experimental import pallas as pl
from jax.experimental.pallas import tpu as pltpu
from jax.experimental.pallas import tpu_sc as plsc

_NC = 2
_NS = 16
_NW = _NC * _NS
_NBUF = 6
_SLACK = 0


def _build(batch, seq, hidden, chunk):
    n_per_w = batch * seq // _NW
    n_ch = n_per_w // chunk
    w_per_b = _NW // batch
    mesh = plsc.VectorSubcoreMesh(core_axis_name="c", subcore_axis_name="s")

    @functools.partial(
        pl.kernel,
        mesh=mesh,
        out_type=jax.ShapeDtypeStruct((batch, seq, hidden), jnp.float32),
        scratch_types=(
            [pltpu.VMEM((n_per_w,), jnp.int32)]
            + [pltpu.VMEM((chunk, hidden), jnp.float32) for _ in range(_NBUF)]
            + [pltpu.SemaphoreType.DMA for _ in range(2 * _NBUF)]
        ),
    )
    def emb(idx_hbm, table_hbm, out_hbm, idx_v, *rest):
        bufs = rest[:_NBUF]
        gsems = rest[_NBUF:2 * _NBUF]
        wsems = rest[2 * _NBUF:]
        wid = lax.axis_index("s") * _NC + lax.axis_index("c")
        bb = wid // w_per_b
        col = (wid % w_per_b) * n_per_w
        pltpu.sync_copy(idx_hbm.at[bb, pl.ds(col, n_per_w)], idx_v)

        def start_gather(i):
            return pltpu.async_copy(table_hbm.at[idx_v.at[pl.ds(i * chunk, chunk)]],
                                    bufs[i % _NBUF], gsems[i % _NBUF])

        lookahead = _NBUF - 1 - _SLACK
        gathers = {j: start_gather(j) for j in range(min(lookahead, n_ch))}
        writebacks = {}
        for i in range(n_ch):
            b = i % _NBUF
            gathers.pop(i).wait()
            j = i + lookahead
            if j < n_ch:
                if i - 1 - _SLACK in writebacks:
                    writebacks.pop(i - 1 - _SLACK).wait()
                gathers[j] = start_gather(j)
            writebacks[i] = pltpu.async_copy(
                bufs[b], out_hbm.at[bb, pl.ds(col + i * chunk, chunk)], wsems[b])
        for i in sorted(writebacks):
            writebacks[i].wait()

    return emb


def kernel(input, word_embeddings):
    b, s = input.shape
    v, d = word_embeddings.shape
    idx = input.astype(jnp.int32)
    return _build(b, s, d, 16)(idx, word_embeddings)
